# baseline (speedup 1.0000x reference)
import functools
import numpy as np
import jax
import jax.numpy as jnp
from jax import lax
from jax.experimental import pallas as pl
from jax.experimental.pallas import tpu as pltpu
from jax.experimental.pallas import tpu_sc as plsc

HIDDEN = 128
NDIM = 3
K = 16
SB = 256
CHUNK = 2048
NSEG = 4
BIGF = 1e30
PADC = 100.0

SC_NC = 2
SC_NS = 16
SC_NW = SC_NC * SC_NS


def _sc_gather(table, idx):
    B = idx.shape[0]
    b_per_w = B // SC_NW
    mesh = plsc.VectorSubcoreMesh(
        core_axis_name="c", subcore_axis_name="s",
        num_cores=SC_NC, num_subcores=SC_NS)

    @functools.partial(
        pl.kernel, mesh=mesh,
        out_type=jax.ShapeDtypeStruct((B, 16), jnp.float32),
        scratch_types=[
            pltpu.VMEM((b_per_w,), jnp.int32),
            pltpu.VMEM((b_per_w, 16), jnp.float32),
            pltpu.SemaphoreType.DMA,
        ],
        compiler_params=pltpu.CompilerParams(use_tc_tiling_on_sc=False),
    )
    def gk(table_hbm, idx_hbm, out_hbm, idx_v, rows_v, sem):
        wid = lax.axis_index("s") * SC_NC + lax.axis_index("c")
        base = wid * b_per_w
        pltpu.sync_copy(idx_hbm.at[pl.ds(base, b_per_w)], idx_v)
        pltpu.async_copy(table_hbm.at[idx_v], rows_v, sem).wait()
        pltpu.sync_copy(rows_v, out_hbm.at[pl.ds(base, b_per_w)])

    return gk(table, idx)


def _knn_body(sup_ref, posT_ref, out_ref, vals_ref, idxs_ref, e_ref):
    j = pl.program_id(1)
    nj = pl.num_programs(1)

    @pl.when(j == 0)
    def _init():
        vals_ref[...] = jnp.full((SB, K), BIGF, jnp.float32)
        idxs_ref[...] = jnp.zeros((SB, K), jnp.int32)

    sup = sup_ref[...]
    pts = posT_ref[...]
    pn = jnp.sum(pts * pts, axis=0, keepdims=True)
    sp = jnp.dot(sup, pts, preferred_element_type=jnp.float32)
    e = pn - 2.0 * sp
    e_ref[...] = e

    base = j * CHUNK
    SEG = CHUNK // NSEG
    lane = lax.broadcasted_iota(jnp.int32, (SB, SEG), 1)
    col = lax.broadcasted_iota(jnp.int32, (SB, K), 1)

    rmax0 = jnp.max(vals_ref[...], axis=1, keepdims=True)
    cnt = jnp.zeros((SB, 1), jnp.int32)
    for g in range(NSEG):
        cg = jnp.sum((e[:, g * SEG:(g + 1) * SEG] < rmax0).astype(jnp.int32),
                     axis=1, keepdims=True)
        cnt = jnp.maximum(cnt, cg)
    niter = jnp.max(jnp.minimum(cnt, jnp.int32(K)))

    def _extract(_, carry):
        vals, idxs = carry
        ec = e_ref[...]
        cms, cis = [], []
        for g in range(NSEG):
            seg = ec[:, g * SEG:(g + 1) * SEG]
            cm = jnp.min(seg, axis=1, keepdims=True)
            ci = jnp.min(jnp.where(seg == cm, lane, jnp.int32(SEG)),
                         axis=1, keepdims=True) + (g * SEG)
            cms.append(cm)
            cis.append(ci)
        for g in range(NSEG):
            rmax = jnp.max(vals, axis=1, keepdims=True)
            rpos = jnp.min(jnp.where(vals == rmax, col, jnp.int32(K)),
                           axis=1, keepdims=True)
            rep = (cms[g] < rmax) & (col == rpos)
            vals = jnp.where(rep, cms[g], vals)
            idxs = jnp.where(rep, base + cis[g], idxs)
        flane = lax.broadcasted_iota(jnp.int32, (SB, CHUNK), 1)
        hit = flane == cis[0]
        for g in range(1, NSEG):
            hit = hit | (flane == cis[g])
        e_ref[...] = jnp.where(hit, BIGF, ec)
        return vals, idxs

    vals, idxs = lax.fori_loop(0, niter, _extract,
                               (vals_ref[...], idxs_ref[...]))
    vals_ref[...] = vals
    idxs_ref[...] = idxs

    @pl.when(j == nj - 1)
    def _fin():
        out_ref[...] = idxs_ref[...]


def _knn_call(sup8, posT8, n_blocks, n_chunks, interpret=False):
    return pl.pallas_call(
        _knn_body,
        grid=(n_blocks, n_chunks),
        in_specs=[
            pl.BlockSpec((SB, 8), lambda i, j: (i, 0)),
            pl.BlockSpec((8, CHUNK), lambda i, j: (0, j)),
        ],
        out_specs=pl.BlockSpec((SB, K), lambda i, j: (i, 0)),
        out_shape=jax.ShapeDtypeStruct((n_blocks * SB, K), jnp.int32),
        scratch_shapes=[
            pltpu.VMEM((SB, K), jnp.float32),
            pltpu.VMEM((SB, K), jnp.int32),
            pltpu.VMEM((SB, CHUNK), jnp.float32),
        ],
        compiler_params=pltpu.CompilerParams(
            dimension_semantics=("parallel", "arbitrary")),
        interpret=interpret,
    )(sup8, posT8)


_LOG2_1E4 = float(np.log2(10000.0))
_HALF_PI = float(np.pi / 2)


def _embed_consts_msg():
    half = 16
    q = lax.broadcasted_iota(jnp.int32, (1, HIDDEN), 1)
    coord = q // 32
    f = (q % 32) % half
    is_cos = (q % 32) >= half
    omega = jnp.exp2(f.astype(jnp.float32) * (-_LOG2_1E4 / half))
    shift = jnp.where(is_cos, _HALF_PI, 0.0)
    return coord, omega, shift


def _embed_consts_sup():
    half = 21
    q = lax.broadcasted_iota(jnp.int32, (1, HIDDEN), 1)
    coord = jnp.minimum(q // 42, 2)
    f = (q % 42) % half
    is_cos = (q % 42) >= half
    omega = jnp.where(q < 126,
                      jnp.exp2(f.astype(jnp.float32) * (-_LOG2_1E4 / half)),
                      0.0)
    shift = jnp.where(is_cos, _HALF_PI, 0.0)
    valid = (q < 126).astype(jnp.float32)
    return coord, omega, shift, valid


def _sincos_select(cols, coord_sel, omega, shift, valid=None):
    M = cols[0].shape[0]
    big = cols[0] * 0.0
    c = jnp.broadcast_to(coord_sel, (M, HIDDEN))
    big = jnp.where(c == 0, cols[0], jnp.where(c == 1, cols[1], cols[2]))
    if len(cols) > 3:
        big = jnp.where(c == 3, cols[3], big)
    emb = jnp.sin(big * omega + shift)
    if valid is not None:
        emb = emb * valid
    return emb


def _msg_body(src_ref, sup_ref, W1_ref, b1_ref, W2_ref, b2_ref,
              Wp_ref, bp_ref, out_ref):
    SBK = SB * K
    src = src_ref[...]
    sup = sup_ref[...]

    r0 = lax.broadcasted_iota(jnp.int32, (SBK, SB), 0) // K
    r1 = lax.broadcasted_iota(jnp.int32, (SBK, SB), 1)
    R = (r0 == r1).astype(jnp.float32)
    dst = jnp.dot(R, sup, preferred_element_type=jnp.float32)

    diff = dst - src
    mag = jnp.sqrt(jnp.sum(diff * diff, axis=1, keepdims=True))

    mc, mo, ms = _embed_consts_msg()
    x = _sincos_select(
        [diff[:, 0:1], diff[:, 1:2], diff[:, 2:3], mag],
        mc, mo, ms)

    h = jnp.dot(x, W1_ref[...], preferred_element_type=jnp.float32) + b1_ref[...]
    h = 0.5 * h * (1.0 + lax.erf(h * np.float32(np.sqrt(0.5))))
    h = jnp.dot(h, W2_ref[...], preferred_element_type=jnp.float32) + b2_ref[...]

    mean = lax.dot_general(R, h, (((0,), (0,)), ((), ())),
                           preferred_element_type=jnp.float32) * (1.0 / K)

    sc_, so_, ss_, sv_ = _embed_consts_sup()
    semb = _sincos_select(
        [sup[:, 0:1], sup[:, 1:2], sup[:, 2:3]],
        sc_, so_, ss_, sv_)

    Wp = Wp_ref[...]
    out = (jnp.dot(mean, Wp[:HIDDEN, :], preferred_element_type=jnp.float32)
           + jnp.dot(semb, Wp[HIDDEN:, :], preferred_element_type=jnp.float32)
           + bp_ref[...])
    out_ref[...] = out


def _msg_call(src16, sup16, W1, b1, W2, b2, Wp, bp, n_blocks, interpret=False):
    full = lambda shape: pl.BlockSpec(shape, lambda i: tuple(0 for _ in shape))
    return pl.pallas_call(
        _msg_body,
        grid=(n_blocks,),
        in_specs=[
            pl.BlockSpec((SB * K, 16), lambda i: (i, 0)),
            pl.BlockSpec((SB, 16), lambda i: (i, 0)),
            full((HIDDEN, HIDDEN)),
            full((1, HIDDEN)),
            full((HIDDEN, HIDDEN)),
            full((1, HIDDEN)),
            full((2 * HIDDEN, HIDDEN)),
            full((1, HIDDEN)),
        ],
        out_specs=pl.BlockSpec((SB, HIDDEN), lambda i: (i, 0)),
        out_shape=jax.ShapeDtypeStruct((n_blocks * SB, HIDDEN), jnp.float32),
        compiler_params=pltpu.CompilerParams(
            dimension_semantics=("parallel",)),
        interpret=interpret,
    )(src16, sup16, W1, b1, W2, b2, Wp, bp)


def kernel(input_pos, supernode_idx, W1, b1, W2, b2, Wp, bp):
    pos = input_pos[..., :NDIM].astype(jnp.float32)
    N = pos.shape[0]
    S = supernode_idx.shape[0]
    n_blocks = S // SB
    n_chunks = -(-N // CHUNK)
    NPAD = n_chunks * CHUNK

    sidx = supernode_idx.astype(jnp.int32)

    pos16 = jnp.zeros((N, 16), jnp.float32).at[:, :NDIM].set(pos)
    posT8 = jnp.full((8, NPAD), 0.0, jnp.float32)
    posT8 = posT8.at[:NDIM, :N].set(pos.T)
    posT8 = posT8.at[:NDIM, N:].set(PADC)

    sup16 = _sc_gather(pos16, sidx)
    knn = _knn_call(sup16[:, :8], posT8, n_blocks, n_chunks)
    src16 = _sc_gather(pos16, knn.reshape(-1))
    out = _msg_call(src16, sup16, W1, b1.reshape(1, -1), W2,
                    b2.reshape(1, -1), Wp, bp.reshape(1, -1), n_blocks)
    return out.reshape(1, S, HIDDEN)


# speedup vs baseline: 1.5649x; 1.5649x over previous
import functools
import numpy as np
import jax
import jax.numpy as jnp
from jax import lax
from jax.experimental import pallas as pl
from jax.experimental.pallas import tpu as pltpu
from jax.experimental.pallas import tpu_sc as plsc

HIDDEN = 128
NDIM = 3
K = 16
SB = 256
CHUNK = 2048
BIGF = 1e30
PADC = 100.0

SC_NC = 2
SC_NS = 16
SC_NW = SC_NC * SC_NS


def _sc_gather(table, idx):
    B = idx.shape[0]
    b_per_w = B // SC_NW
    mesh = plsc.VectorSubcoreMesh(
        core_axis_name="c", subcore_axis_name="s",
        num_cores=SC_NC, num_subcores=SC_NS)

    @functools.partial(
        pl.kernel, mesh=mesh,
        out_type=jax.ShapeDtypeStruct((B, 16), jnp.float32),
        scratch_types=[
            pltpu.VMEM((b_per_w,), jnp.int32),
            pltpu.VMEM((b_per_w, 16), jnp.float32),
            pltpu.SemaphoreType.DMA,
        ],
        compiler_params=pltpu.CompilerParams(use_tc_tiling_on_sc=False),
    )
    def gk(table_hbm, idx_hbm, out_hbm, idx_v, rows_v, sem):
        wid = lax.axis_index("s") * SC_NC + lax.axis_index("c")
        base = wid * b_per_w
        pltpu.sync_copy(idx_hbm.at[pl.ds(base, b_per_w)], idx_v)
        pltpu.async_copy(table_hbm.at[idx_v], rows_v, sem).wait()
        pltpu.sync_copy(rows_v, out_hbm.at[pl.ds(base, b_per_w)])

    return gk(table, idx)


def _knn_body(sup_ref, posT_ref, out_ref, vals_ref, idxs_ref, e_ref):
    j = pl.program_id(1)
    nj = pl.num_programs(1)

    @pl.when(j == 0)
    def _init():
        vals_ref[...] = jnp.full((SB, K), BIGF, jnp.float32)
        idxs_ref[...] = jnp.zeros((SB, K), jnp.int32)

    sup = sup_ref[...]
    pts = posT_ref[...]
    pn = jnp.sum(pts * pts, axis=0, keepdims=True)
    sp = jnp.dot(sup, pts, preferred_element_type=jnp.float32)
    e = pn - 2.0 * sp
    e_ref[...] = e

    base = j * CHUNK
    lane = lax.broadcasted_iota(jnp.int32, (SB, CHUNK), 1)
    col = lax.broadcasted_iota(jnp.int32, (SB, K), 1)

    rmax0 = jnp.max(vals_ref[...], axis=1, keepdims=True)
    cnt = jnp.sum((e < rmax0).astype(jnp.int32), axis=1, keepdims=True)
    niter = jnp.max(jnp.minimum(cnt, jnp.int32(K)))

    def _extract(_, carry):
        vals, idxs = carry
        ec = e_ref[...]
        cmin = jnp.min(ec, axis=1, keepdims=True)
        cidx = jnp.min(jnp.where(ec == cmin, lane, jnp.int32(CHUNK)),
                       axis=1, keepdims=True)
        rmax = jnp.max(vals, axis=1, keepdims=True)
        rpos = jnp.min(jnp.where(vals == rmax, col, jnp.int32(K)),
                       axis=1, keepdims=True)
        better = cmin < rmax
        rep = better & (col == rpos)
        vals = jnp.where(rep, cmin, vals)
        idxs = jnp.where(rep, base + cidx, idxs)
        e_ref[...] = jnp.where(lane == cidx, BIGF, ec)
        return vals, idxs

    vals, idxs = lax.fori_loop(0, niter, _extract,
                               (vals_ref[...], idxs_ref[...]))
    vals_ref[...] = vals
    idxs_ref[...] = idxs

    @pl.when(j == nj - 1)
    def _fin():
        out_ref[...] = idxs_ref[...]


def _knn_call(sup8, posT8, n_blocks, n_chunks, interpret=False):
    return pl.pallas_call(
        _knn_body,
        grid=(n_blocks, n_chunks),
        in_specs=[
            pl.BlockSpec((SB, 8), lambda i, j: (i, 0)),
            pl.BlockSpec((8, CHUNK), lambda i, j: (0, j)),
        ],
        out_specs=pl.BlockSpec((SB, K), lambda i, j: (i, 0)),
        out_shape=jax.ShapeDtypeStruct((n_blocks * SB, K), jnp.int32),
        scratch_shapes=[
            pltpu.VMEM((SB, K), jnp.float32),
            pltpu.VMEM((SB, K), jnp.int32),
            pltpu.VMEM((SB, CHUNK), jnp.float32),
        ],
        compiler_params=pltpu.CompilerParams(
            dimension_semantics=("parallel", "arbitrary")),
        interpret=interpret,
    )(sup8, posT8)


_LOG2_1E4 = float(np.log2(10000.0))
_HALF_PI = float(np.pi / 2)


def _embed_consts_msg():
    half = 16
    q = lax.broadcasted_iota(jnp.int32, (1, HIDDEN), 1)
    coord = q // 32
    f = (q % 32) % half
    is_cos = (q % 32) >= half
    omega = jnp.exp2(f.astype(jnp.float32) * (-_LOG2_1E4 / half))
    shift = jnp.where(is_cos, _HALF_PI, 0.0)
    return coord, omega, shift


def _embed_consts_sup():
    half = 21
    q = lax.broadcasted_iota(jnp.int32, (1, HIDDEN), 1)
    coord = jnp.minimum(q // 42, 2)
    f = (q % 42) % half
    is_cos = (q % 42) >= half
    omega = jnp.where(q < 126,
                      jnp.exp2(f.astype(jnp.float32) * (-_LOG2_1E4 / half)),
                      0.0)
    shift = jnp.where(is_cos, _HALF_PI, 0.0)
    valid = (q < 126).astype(jnp.float32)
    return coord, omega, shift, valid


def _sincos_select(cols, coord_sel, omega, shift, valid=None):
    M = cols[0].shape[0]
    big = cols[0] * 0.0
    c = jnp.broadcast_to(coord_sel, (M, HIDDEN))
    big = jnp.where(c == 0, cols[0], jnp.where(c == 1, cols[1], cols[2]))
    if len(cols) > 3:
        big = jnp.where(c == 3, cols[3], big)
    emb = jnp.sin(big * omega + shift)
    if valid is not None:
        emb = emb * valid
    return emb


def _msg_body(src_ref, sup_ref, W1_ref, b1_ref, W2_ref, b2_ref,
              Wp_ref, bp_ref, out_ref):
    SBK = SB * K
    src = src_ref[...]
    sup = sup_ref[...]

    r0 = lax.broadcasted_iota(jnp.int32, (SBK, SB), 0) // K
    r1 = lax.broadcasted_iota(jnp.int32, (SBK, SB), 1)
    R = (r0 == r1).astype(jnp.float32)
    dst = jnp.dot(R, sup, preferred_element_type=jnp.float32)

    diff = dst - src
    mag = jnp.sqrt(jnp.sum(diff * diff, axis=1, keepdims=True))

    mc, mo, ms = _embed_consts_msg()
    x = _sincos_select(
        [diff[:, 0:1], diff[:, 1:2], diff[:, 2:3], mag],
        mc, mo, ms)

    h = jnp.dot(x, W1_ref[...], preferred_element_type=jnp.float32) + b1_ref[...]
    h = 0.5 * h * (1.0 + lax.erf(h * np.float32(np.sqrt(0.5))))
    h = jnp.dot(h, W2_ref[...], preferred_element_type=jnp.float32) + b2_ref[...]

    mean = lax.dot_general(R, h, (((0,), (0,)), ((), ())),
                           preferred_element_type=jnp.float32) * (1.0 / K)

    sc_, so_, ss_, sv_ = _embed_consts_sup()
    semb = _sincos_select(
        [sup[:, 0:1], sup[:, 1:2], sup[:, 2:3]],
        sc_, so_, ss_, sv_)

    Wp = Wp_ref[...]
    out = (jnp.dot(mean, Wp[:HIDDEN, :], preferred_element_type=jnp.float32)
           + jnp.dot(semb, Wp[HIDDEN:, :], preferred_element_type=jnp.float32)
           + bp_ref[...])
    out_ref[...] = out


def _msg_call(src16, sup16, W1, b1, W2, b2, Wp, bp, n_blocks, interpret=False):
    full = lambda shape: pl.BlockSpec(shape, lambda i: tuple(0 for _ in shape))
    return pl.pallas_call(
        _msg_body,
        grid=(n_blocks,),
        in_specs=[
            pl.BlockSpec((SB * K, 16), lambda i: (i, 0)),
            pl.BlockSpec((SB, 16), lambda i: (i, 0)),
            full((HIDDEN, HIDDEN)),
            full((1, HIDDEN)),
            full((HIDDEN, HIDDEN)),
            full((1, HIDDEN)),
            full((2 * HIDDEN, HIDDEN)),
            full((1, HIDDEN)),
        ],
        out_specs=pl.BlockSpec((SB, HIDDEN), lambda i: (i, 0)),
        out_shape=jax.ShapeDtypeStruct((n_blocks * SB, HIDDEN), jnp.float32),
        compiler_params=pltpu.CompilerParams(
            dimension_semantics=("parallel",)),
        interpret=interpret,
    )(src16, sup16, W1, b1, W2, b2, Wp, bp)


def kernel(input_pos, supernode_idx, W1, b1, W2, b2, Wp, bp):
    pos = input_pos[..., :NDIM].astype(jnp.float32)
    N = pos.shape[0]
    S = supernode_idx.shape[0]
    n_blocks = S // SB
    n_chunks = -(-N // CHUNK)
    NPAD = n_chunks * CHUNK

    sidx = supernode_idx.astype(jnp.int32)

    pos16 = jnp.zeros((N, 16), jnp.float32).at[:, :NDIM].set(pos)
    posT8 = jnp.full((8, NPAD), 0.0, jnp.float32)
    posT8 = posT8.at[:NDIM, :N].set(pos.T)
    posT8 = posT8.at[:NDIM, N:].set(PADC)

    sup16 = _sc_gather(pos16, sidx)
    knn = _knn_call(sup16[:, :8], posT8, n_blocks, n_chunks)
    src16 = _sc_gather(pos16, knn.reshape(-1))
    out = _msg_call(src16, sup16, W1, b1.reshape(1, -1), W2,
                    b2.reshape(1, -1), Wp, bp.reshape(1, -1), n_blocks)
    return out.reshape(1, S, HIDDEN)
